# Initial kernel scaffold; baseline (speedup 1.0000x reference)
import jax
import jax.numpy as jnp
from jax.experimental import pallas as pl


def kernel(X, W1_0, b1_0, W1_1, b1_1, W1_2, b1_2, Wsc1, W2_0, b2_0, W2_1, b2_1, W2_2, b2_2, Wsc2, Wp1, bp1, Wp2, bp2, Wm, bm, Wout, bout):
    raise NotImplementedError("write your pallas kernel here")


# speedup vs baseline: 1.2880x; 1.2880x over previous
import jax
import jax.numpy as jnp
from jax.experimental import pallas as pl
from jax.experimental.pallas import tpu as pltpu

_B, _F_IN, _N, _K = 32, 16, 128, 16
_C = 32
_HID, _MLP_DIM, _NCLS = 32, 128, 2
_BIG = 1e30


def _knn_onehots(pts):
    ii = jax.lax.broadcasted_iota(jnp.int32, (_N, _N), 0)
    jj = jax.lax.broadcasted_iota(jnp.int32, (_N, _N), 1)
    dist = jnp.zeros((_N, _N), jnp.float32)
    for c in range(3):
        d = pts[:, c:c + 1] - pts[:, c:c + 1].T
        dist = dist + d * d
    dist = jnp.where(ii == jj, _BIG, dist)
    ohs = []
    for _ in range(_K):
        rowmin = jnp.min(dist, axis=1, keepdims=True)
        ismin = dist <= rowmin
        idx = jnp.min(jnp.where(ismin, jj, _N), axis=1, keepdims=True)
        oh = jj == idx
        ohs.append(oh.astype(jnp.float32))
        dist = jnp.where(oh, _BIG, dist)
    return jnp.concatenate(ohs, axis=0)


def _edge_conv(Xt, W0t, b0, W1t, b1, W2t, b2, Wsct):
    O = _knn_onehots(Xt[:, 0:3])
    Xnn = jnp.dot(O, Xt, preferred_element_type=jnp.float32)
    Xc = jnp.concatenate([Xt] * _K, axis=0)
    H = jnp.concatenate([Xnn - Xc, Xc], axis=1)
    h = jnp.maximum(jnp.dot(H, W0t, preferred_element_type=jnp.float32) + b0, 0.0)
    h = jnp.maximum(jnp.dot(h, W1t, preferred_element_type=jnp.float32) + b1, 0.0)
    h = jnp.maximum(jnp.dot(h, W2t, preferred_element_type=jnp.float32) + b2, 0.0)
    acc = jnp.zeros((_N, _C), jnp.float32)
    for k in range(_K):
        acc = acc + h[k * _N:(k + 1) * _N, :]
    Hp = acc * (1.0 / _K)
    sc = jnp.dot(Xt, Wsct, preferred_element_type=jnp.float32)
    return jnp.maximum(Hp + sc, 0.0)


def _body(xt_ref,
          W10t, b10, W11t, b11, W12t, b12, Wsc1t,
          W20t, b20, W21t, b21, W22t, b22, Wsc2t,
          Wp1at, Wp1bt, bp1, wp2, bp2, Wmt, bm, Woutt, bout,
          pred_ref, ev_ref):
    Xt = xt_ref[0]
    f1 = _edge_conv(Xt, W10t[...], b10[...], W11t[...], b11[...],
                    W12t[...], b12[...], Wsc1t[...])
    f2 = _edge_conv(f1, W20t[...], b20[...], W21t[...], b21[...],
                    W22t[...], b22[...], Wsc2t[...])
    At = jnp.dot(f2, Wp1at[...], preferred_element_type=jnp.float32)
    Bt = jnp.dot(f2, Wp1bt[...], preferred_element_type=jnp.float32) + bp1[...]
    AtT = At.T
    w2 = wp2[...]
    E = jnp.zeros((_N, _N), jnp.float32)
    for c in range(_HID):
        term = jnp.maximum(AtT[c:c + 1, :] + Bt[:, c:c + 1], 0.0)
        E = E + w2[0:1, c:c + 1] * term
    ev = E + E.T + 2.0 * bp2[...]
    ev_ref[0] = ev
    pooled = jnp.mean(f2, axis=0, keepdims=True)
    h2 = jnp.maximum(jnp.dot(pooled, Wmt[...], preferred_element_type=jnp.float32)
                     + bm[...], 0.0)
    pred = jnp.dot(h2, Woutt[...], preferred_element_type=jnp.float32) + bout[...]
    pred_ref[0] = pred


def kernel(X, W1_0, b1_0, W1_1, b1_1, W1_2, b1_2, Wsc1,
           W2_0, b2_0, W2_1, b2_1, W2_2, b2_2, Wsc2,
           Wp1, bp1, Wp2, bp2, Wm, bm, Wout, bout):
    Xt = jnp.transpose(X, (0, 2, 1))
    row = lambda v: v.reshape(1, -1)
    ws = [
        W1_0.T, row(b1_0), W1_1.T, row(b1_1), W1_2.T, row(b1_2), Wsc1.T,
        W2_0.T, row(b2_0), W2_1.T, row(b2_1), W2_2.T, row(b2_2), Wsc2.T,
        Wp1[:, :_C].T, Wp1[:, _C:].T, row(bp1), Wp2, bp2.reshape(1, 1),
        Wm.T, row(bm), Wout.T, row(bout),
    ]
    in_specs = [pl.BlockSpec((1, _N, _F_IN), lambda b: (b, 0, 0))]
    for w in ws:
        in_specs.append(pl.BlockSpec(w.shape, lambda b, nd=w.ndim: (0,) * nd))
    out_shape = [
        jax.ShapeDtypeStruct((_B, 1, _NCLS), jnp.float32),
        jax.ShapeDtypeStruct((_B, _N, _N), jnp.float32),
    ]
    out_specs = [
        pl.BlockSpec((1, 1, _NCLS), lambda b: (b, 0, 0)),
        pl.BlockSpec((1, _N, _N), lambda b: (b, 0, 0)),
    ]
    pred3, ev = pl.pallas_call(
        _body,
        grid=(_B,),
        in_specs=in_specs,
        out_specs=out_specs,
        out_shape=out_shape,
        compiler_params=pltpu.CompilerParams(
            dimension_semantics=("arbitrary",),
        ),
    )(Xt, *ws)
    return pred3.reshape(_B, _NCLS), ev


# speedup vs baseline: 4.1942x; 3.2563x over previous
import jax
import jax.numpy as jnp
from jax.experimental import pallas as pl
from jax.experimental.pallas import tpu as pltpu

_B, _F_IN, _N, _K = 32, 16, 128, 16
_C = 32
_HID, _MLP_DIM, _NCLS = 32, 128, 2
_BIG = 1e30
_BPP = 2


def _knn_onehots(pts):
    ii = jax.lax.broadcasted_iota(jnp.int32, (_N, _N), 0)
    jj = jax.lax.broadcasted_iota(jnp.int32, (_N, _N), 1)
    dist = jnp.zeros((_N, _N), jnp.float32)
    for c in range(3):
        row = pts[c:c + 1, :]
        d = row.T - row
        dist = dist + d * d
    dist = jnp.where(ii == jj, _BIG, dist)
    ohs = []
    for _ in range(_K):
        colmin = jnp.min(dist, axis=0, keepdims=True)
        idx = jnp.min(jnp.where(dist <= colmin, ii, _N), axis=0,
                      keepdims=True)
        oh = ii == idx
        ohs.append(oh.astype(jnp.float32))
        dist = jnp.where(oh, _BIG, dist)
    return jnp.concatenate(ohs, axis=1)


def _edge_conv(Xf, W0, b0, W1, b1, W2, b2, Wsc):
    OT = _knn_onehots(Xf[0:3, :])
    W0a = W0[:, : Xf.shape[0]]
    W0b = W0[:, Xf.shape[0]:]
    P = jnp.dot(W0a, Xf, preferred_element_type=jnp.float32)
    Q = jnp.dot(W0b - W0a, Xf, preferred_element_type=jnp.float32) + b0
    Pg = jnp.dot(P, OT, preferred_element_type=jnp.float32)
    Qt = jnp.concatenate([Q] * _K, axis=1)
    h = jnp.maximum(Pg + Qt, 0.0)
    h = jnp.maximum(jnp.dot(W1, h, preferred_element_type=jnp.float32) + b1, 0.0)
    h = jnp.maximum(jnp.dot(W2, h, preferred_element_type=jnp.float32) + b2, 0.0)
    acc = jnp.zeros((_C, _N), jnp.float32)
    for k in range(_K):
        acc = acc + h[:, k * _N:(k + 1) * _N]
    sc = jnp.dot(Wsc, Xf, preferred_element_type=jnp.float32)
    return jnp.maximum(acc * (1.0 / _K) + sc, 0.0)


def _one_batch(Xf, W10, b10, W11, b11, W12, b12, Wsc1,
               W20, b20, W21, b21, W22, b22, Wsc2,
               Wp1a, Wp1b, bp1, wp2, bp2, Wm, bm, Wout, bout):
    f1 = _edge_conv(Xf, W10, b10, W11, b11, W12, b12, Wsc1)
    f2 = _edge_conv(f1, W20, b20, W21, b21, W22, b22, Wsc2)
    A = jnp.dot(Wp1a, f2, preferred_element_type=jnp.float32)
    Bm = jnp.dot(Wp1b, f2, preferred_element_type=jnp.float32) + bp1
    BT = Bm.T
    E = jnp.zeros((_N, _N), jnp.float32)
    for c in range(_HID):
        term = jnp.maximum(A[c:c + 1, :] + BT[:, c:c + 1], 0.0)
        E = E + wp2[0:1, c:c + 1] * term
    ev = E + E.T + 2.0 * bp2
    pooled = jnp.mean(f2, axis=1, keepdims=True).T
    h2 = jnp.maximum(jnp.dot(pooled, Wm, preferred_element_type=jnp.float32)
                     + bm, 0.0)
    pred = jnp.dot(h2, Wout, preferred_element_type=jnp.float32) + bout
    return pred, ev


def _body(x_ref,
          W10, b10, W11, b11, W12, b12, Wsc1,
          W20, b20, W21, b21, W22, b22, Wsc2,
          Wp1a, Wp1b, bp1, wp2, bp2, Wm, bm, Wout, bout,
          pred_ref, ev_ref):
    ws = (W10[...], b10[...], W11[...], b11[...], W12[...], b12[...],
          Wsc1[...],
          W20[...], b20[...], W21[...], b21[...], W22[...], b22[...],
          Wsc2[...],
          Wp1a[...], Wp1b[...], bp1[...], wp2[...], bp2[...],
          Wm[...], bm[...], Wout[...], bout[...])
    for i in range(_BPP):
        pred, ev = _one_batch(x_ref[i], *ws)
        ev_ref[i] = ev
        pred_ref[i] = pred


def kernel(X, W1_0, b1_0, W1_1, b1_1, W1_2, b1_2, Wsc1,
           W2_0, b2_0, W2_1, b2_1, W2_2, b2_2, Wsc2,
           Wp1, bp1, Wp2, bp2, Wm, bm, Wout, bout):
    col = lambda v: v.reshape(-1, 1)
    ws = [
        W1_0, col(b1_0), W1_1, col(b1_1), W1_2, col(b1_2), Wsc1,
        W2_0, col(b2_0), W2_1, col(b2_1), W2_2, col(b2_2), Wsc2,
        Wp1[:, :_C], Wp1[:, _C:], col(bp1), Wp2, bp2.reshape(1, 1),
        Wm.T, bm.reshape(1, -1), Wout.T, bout.reshape(1, -1),
    ]
    in_specs = [pl.BlockSpec((_BPP, _F_IN, _N), lambda b: (b, 0, 0))]
    for w in ws:
        in_specs.append(pl.BlockSpec(w.shape, lambda b, nd=w.ndim: (0,) * nd))
    out_shape = [
        jax.ShapeDtypeStruct((_B, 1, _NCLS), jnp.float32),
        jax.ShapeDtypeStruct((_B, _N, _N), jnp.float32),
    ]
    out_specs = [
        pl.BlockSpec((_BPP, 1, _NCLS), lambda b: (b, 0, 0)),
        pl.BlockSpec((_BPP, _N, _N), lambda b: (b, 0, 0)),
    ]
    pred3, ev = pl.pallas_call(
        _body,
        grid=(_B // _BPP,),
        in_specs=in_specs,
        out_specs=out_specs,
        out_shape=out_shape,
        compiler_params=pltpu.CompilerParams(
            dimension_semantics=("arbitrary",),
        ),
    )(X, *ws)
    return pred3.reshape(_B, _NCLS), ev
